# baseline (speedup 1.0000x reference)
import functools

import jax
import jax.numpy as jnp
from jax import lax
from jax.experimental import pallas as pl
from jax.experimental.pallas import tpu as pltpu
from jax.experimental.pallas import tpu_sc as plsc

N = 10000
E = 320000
D = 128

NW = 32
SEG = 320
LANES = 16
ECHUNK = 3200
NCHUNK = E // ECHUNK
SUBCAP = 1024
CAP = LANES * SUBCAP
GB = 256
LCAP = CAP + GB
PACK = 512
DW = D // 2
NP = N // 2
NPP = 5120
SLAB = NPP // LANES

_sc_params = pltpu.CompilerParams(needs_layout_passes=False)


def _sc_mesh():
    return plsc.VectorSubcoreMesh(core_axis_name="c", subcore_axis_name="s")


def _wid():
    return lax.axis_index("s") * 2 + lax.axis_index("c")


def _make_preprocess():
    @functools.partial(
        pl.kernel,
        mesh=_sc_mesh(),
        compiler_params=_sc_params,
        out_type=(
            jax.ShapeDtypeStruct((NW * LCAP,), jnp.int32),
            jax.ShapeDtypeStruct((NW * LCAP,), jnp.int32),
            jax.ShapeDtypeStruct((NW * LANES,), jnp.int32),
        ),
        scratch_types=[
            pltpu.VMEM((ECHUNK,), jnp.int32),
            pltpu.VMEM((ECHUNK,), jnp.int32),
            pltpu.VMEM((ECHUNK,), jnp.int32),
            pltpu.VMEM((ECHUNK,), jnp.int32),
            pltpu.VMEM((CAP,), jnp.int32),
            pltpu.VMEM((LCAP,), jnp.int32),
            pltpu.VMEM((LCAP,), jnp.int32),
            pltpu.VMEM((LANES,), jnp.int32),
            pltpu.VMEM_SHARED((4, ECHUNK), jnp.int32),
            pltpu.SemaphoreType.DMA,
            pltpu.SemaphoreType.DMA,
        ],
    )
    def preprocess(src_hbm, dst_hbm, msrc_hbm, mrel_hbm, cnt_hbm,
                   src0, dst0, src1, dst1, packed, fsrc, frel, cntv,
                   eshared, semA, semB):
        wid = _wid()
        base = wid * SEG
        is0 = lax.axis_index("s") == 0
        lane = lax.iota(jnp.int32, LANES)
        ones = jnp.ones((LANES,), jnp.int32)

        padv = jnp.full((LANES,), SEG, jnp.int32)
        zerov = jnp.zeros((LANES,), jnp.int32)

        def fill_packed(i, _):
            packed[pl.ds(i * LANES, LANES)] = padv
            return 0

        lax.fori_loop(0, CAP // LANES, fill_packed, 0, unroll=8)

        def fill_flat(i, _):
            fsrc[pl.ds(i * LANES, LANES)] = zerov
            frel[pl.ds(i * LANES, LANES)] = padv
            return 0

        lax.fori_loop(0, LCAP // LANES, fill_flat, 0, unroll=8)

        def scan_chunk(sbuf, dbuf, counts):
            def vbody(v, counts):
                d = dbuf[pl.ds(v * LANES, LANES)]
                s = sbuf[pl.ds(v * LANES, LANES)]
                rel = d - base
                m = (rel >= 0) & (rel < SEG) & (counts < SUBCAP)
                pos = lane * SUBCAP + counts
                plsc.store_scatter(packed, [pos], s * PACK + rel, mask=m)
                return counts + jnp.where(m, ones, 0)

            return lax.fori_loop(0, ECHUNK // LANES, vbody, counts, unroll=4)

        def stage(c, slot, sem):
            pltpu.async_copy(src_hbm.at[pl.ds(c * ECHUNK, ECHUNK)],
                             eshared.at[2 * slot], sem)
            pltpu.async_copy(dst_hbm.at[pl.ds(c * ECHUNK, ECHUNK)],
                             eshared.at[2 * slot + 1], sem)

        def stage_wait(slot, sem):
            pltpu.make_async_copy(src_hbm.at[pl.ds(0, ECHUNK)],
                                  eshared.at[2 * slot], sem).wait()
            pltpu.make_async_copy(dst_hbm.at[pl.ds(0, ECHUNK)],
                                  eshared.at[2 * slot + 1], sem).wait()

        def fetch_and_scan(slot, sbuf, dbuf, counts):
            pltpu.sync_copy(eshared.at[2 * slot], sbuf)
            pltpu.sync_copy(eshared.at[2 * slot + 1], dbuf)
            return scan_chunk(sbuf, dbuf, counts)

        @pl.when(is0)
        def _():
            stage(0, 0, semA)
            stage_wait(0, semA)

        plsc.subcore_barrier()

        def pair_body(i, counts):
            c0 = 2 * i

            @pl.when(is0)
            def _():
                stage(c0 + 1, 1, semB)

            counts = fetch_and_scan(0, src0, dst0, counts)

            @pl.when(is0)
            def _():
                stage_wait(1, semB)

            plsc.subcore_barrier()

            @pl.when(is0 & (c0 + 2 < NCHUNK))
            def _():
                stage(c0 + 2, 0, semA)

            counts = fetch_and_scan(1, src1, dst1, counts)

            @pl.when(is0 & (c0 + 2 < NCHUNK))
            def _():
                stage_wait(0, semA)

            plsc.subcore_barrier()
            return counts

        counts = lax.fori_loop(0, NCHUNK // 2, pair_body,
                               jnp.zeros((LANES,), jnp.int32))

        total = jnp.int32(0)
        for l in range(LANES):
            cl = counts[l]

            def cbody(i, _, l=l, total=total):
                pv = packed[pl.ds(l * SUBCAP + i * LANES, LANES)]
                s = pv // PACK
                hi = (s >= NP).astype(jnp.int32)
                fsrc[pl.ds(total + i * LANES, LANES)] = s - hi * NP
                frel[pl.ds(total + i * LANES, LANES)] = pv % PACK + hi * PACK
                return 0

            lax.fori_loop(0, (cl + LANES - 1) // LANES, cbody, 0)
            total = total + cl

        pltpu.sync_copy(fsrc, msrc_hbm.at[pl.ds(wid * LCAP, LCAP)])
        pltpu.sync_copy(frel, mrel_hbm.at[pl.ds(wid * LCAP, LCAP)])
        cntv[...] = jnp.full((LANES,), total, jnp.int32)
        pltpu.sync_copy(cntv, cnt_hbm.at[pl.ds(wid * LANES, LANES)])

    return preprocess


def _make_segmax():
    @functools.partial(
        pl.kernel,
        mesh=_sc_mesh(),
        compiler_params=_sc_params,
        out_type=jax.ShapeDtypeStruct((NW * SEG * DW,), jnp.int32),
        scratch_types=[
            pltpu.VMEM(((SEG + 1) * DW,), jnp.int32),
            pltpu.VMEM((LCAP,), jnp.int32),
            pltpu.VMEM((LCAP,), jnp.int32),
            pltpu.VMEM((GB, D), jnp.int32),
            pltpu.VMEM((GB, D), jnp.int32),
            pltpu.VMEM((LANES,), jnp.int32),
            pltpu.SemaphoreType.DMA,
            pltpu.SemaphoreType.DMA,
        ],
    )
    def segmax(t_hbm, msrc_hbm, mrel_hbm, cnt_hbm, agg_hbm,
               acc, idxv, relv, rows0, rows1, cntv, semA, semB):
        wid = _wid()

        zrow = jnp.zeros((LANES,), jnp.int32)

        def zbody(i, _):
            acc[pl.ds(i * LANES, LANES)] = zrow
            return 0

        lax.fori_loop(0, (SEG + 1) * DW // LANES, zbody, 0, unroll=8)

        pltpu.sync_copy(cnt_hbm.at[pl.ds(wid * LANES, LANES)], cntv)
        pltpu.sync_copy(msrc_hbm.at[pl.ds(wid * LCAP, LCAP)], idxv)
        pltpu.sync_copy(mrel_hbm.at[pl.ds(wid * LCAP, LCAP)], relv)
        p = cntv[...][0]
        nbatch = (p + (GB - 1)) // GB

        def start(b, rows, sem):
            return pltpu.async_copy(t_hbm.at[idxv.at[pl.ds(b * GB, GB)]],
                                    rows, sem)

        def wait(rows, sem):
            pltpu.make_async_copy(t_hbm.at[idxv.at[pl.ds(0, GB)]], rows,
                                  sem).wait()

        def process(b, rows):
            def gbody(g, _):
                relvec = relv[pl.ds(b * GB + g * LANES, LANES)]
                for l in range(LANES):
                    rp = relvec[l]
                    a0 = (rp % PACK) * DW
                    ho = (rp // PACK) * DW
                    r0 = g * LANES + l
                    for cb in range(DW // LANES):
                        av = plsc.bitcast(acc[pl.ds(a0 + cb * LANES, LANES)],
                                          jnp.bfloat16)
                        rv = plsc.bitcast(rows[r0, pl.ds(ho + cb * LANES, LANES)],
                                          jnp.bfloat16)
                        acc[pl.ds(a0 + cb * LANES, LANES)] = plsc.bitcast(
                            jnp.maximum(av, rv), jnp.int32)
                return 0

            lax.fori_loop(0, GB // LANES, gbody, 0)

        @pl.when(nbatch > 0)
        def _():
            start(0, rows0, semA)

        def pair_body(i, _):
            b0 = 2 * i
            wait(rows0, semA)

            @pl.when(b0 + 1 < nbatch)
            def _():
                start(b0 + 1, rows1, semB)

            process(b0, rows0)

            @pl.when(b0 + 1 < nbatch)
            def _():
                wait(rows1, semB)

                @pl.when(b0 + 2 < nbatch)
                def _():
                    start(b0 + 2, rows0, semA)

                process(b0 + 1, rows1)

            return 0

        lax.fori_loop(0, (nbatch + 1) // 2, pair_body, 0)
        pltpu.sync_copy(acc.at[pl.ds(0, SEG * DW)],
                        agg_hbm.at[pl.ds(wid * SEG * DW, SEG * DW)])

    return segmax


RB = 1000


def _pack16(ze, zo):
    ue = lax.bitcast_convert_type(ze.astype(jnp.bfloat16), jnp.uint16)
    uo = lax.bitcast_convert_type(zo.astype(jnp.bfloat16), jnp.uint16)
    w = ue.astype(jnp.uint32) | (uo.astype(jnp.uint32) << 16)
    return w.astype(jnp.int32)


def _agg_mlp_words(h, we_ref, wo_ref, be_ref, bo_ref):
    ze = jax.nn.relu(
        jnp.dot(h, we_ref[...], preferred_element_type=jnp.float32) + be_ref[...])
    zo = jax.nn.relu(
        jnp.dot(h, wo_ref[...], preferred_element_type=jnp.float32) + bo_ref[...])
    return _pack16(ze, zo)


def _pre_body(hlo_ref, hhi_ref, we_ref, wo_ref, be_ref, bo_ref, out_ref):
    wlo = _agg_mlp_words(hlo_ref[...], we_ref, wo_ref, be_ref, bo_ref)
    whi = _agg_mlp_words(hhi_ref[...], we_ref, wo_ref, be_ref, bo_ref)
    out_ref[...] = jnp.concatenate([wlo, whi], axis=1)


_half_spec = pl.BlockSpec((RB, D), lambda i: (i, 0))
_mat_spec = pl.BlockSpec((D, DW), lambda i: (0, 0))
_matf_spec = pl.BlockSpec((D, D), lambda i: (0, 0))
_bias_spec = pl.BlockSpec((1, DW), lambda i: (0, 0))
_biasf_spec = pl.BlockSpec((1, D), lambda i: (0, 0))
_lo_spec = pl.BlockSpec((RB, D), lambda i: (i, 0))
_hi_spec = pl.BlockSpec((RB, D), lambda i: (i + NP // RB, 0))


def _split_agg_weights(W_agg, b_agg):
    We, Wo = W_agg[:, 0::2], W_agg[:, 1::2]
    be, bo = b_agg[0::2].reshape(1, DW), b_agg[1::2].reshape(1, DW)
    return We, Wo, be, bo


def _dense_pre(h, W_agg, b_agg):
    We, Wo, be, bo = _split_agg_weights(W_agg, b_agg)
    return pl.pallas_call(
        _pre_body,
        grid=(NP // RB,),
        in_specs=[_lo_spec, _hi_spec, _mat_spec, _mat_spec, _bias_spec,
                  _bias_spec],
        out_specs=pl.BlockSpec((RB, D), lambda i: (i, 0)),
        out_shape=jax.ShapeDtypeStruct((NPP, D), jnp.int32),
    )(h, h, We, Wo, be, bo)


def _half_update(h, agg, wh_ref, wa_ref, b_ref):
    z = (
        jnp.dot(h, wh_ref[...], preferred_element_type=jnp.float32)
        + jnp.dot(agg, wa_ref[...], preferred_element_type=jnp.float32)
        + b_ref[...]
    )
    z = jax.nn.relu(z)
    norm = jnp.sqrt(jnp.sum(z * z, axis=1, keepdims=True))
    return z / jnp.maximum(norm, 1e-12)


def _layer_fused_body(hlo_ref, hhi_ref, alo_ref, ahi_ref, wh_ref, wa_ref,
                      b_ref, we_ref, wo_ref, be_ref, bo_ref,
                      olo_ref, ohi_ref, t_ref):
    hlo = _half_update(hlo_ref[...], alo_ref[...], wh_ref, wa_ref, b_ref)
    hhi = _half_update(hhi_ref[...], ahi_ref[...], wh_ref, wa_ref, b_ref)
    olo_ref[...] = hlo
    ohi_ref[...] = hhi
    wlo = _agg_mlp_words(hlo, we_ref, wo_ref, be_ref, bo_ref)
    whi = _agg_mlp_words(hhi, we_ref, wo_ref, be_ref, bo_ref)
    t_ref[...] = jnp.concatenate([wlo, whi], axis=1)


def _layer_body(hlo_ref, hhi_ref, alo_ref, ahi_ref, wh_ref, wa_ref, b_ref,
                olo_ref, ohi_ref):
    olo_ref[...] = _half_update(hlo_ref[...], alo_ref[...], wh_ref, wa_ref,
                                b_ref)
    ohi_ref[...] = _half_update(hhi_ref[...], ahi_ref[...], wh_ref, wa_ref,
                                b_ref)


def _dense_layer(h_lo, h_hi, agg, W, b, W_agg=None, b_agg=None):
    Wh, Wa = W[:D], W[D:].astype(jnp.bfloat16)
    half = pl.BlockSpec((RB, D), lambda i: (i, 0))
    if W_agg is None:
        return pl.pallas_call(
            _layer_body,
            grid=(NP // RB,),
            in_specs=[half, half, _lo_spec, _hi_spec, _matf_spec, _matf_spec,
                      _biasf_spec],
            out_specs=(half, half),
            out_shape=(jax.ShapeDtypeStruct((NP, D), jnp.float32),
                       jax.ShapeDtypeStruct((NP, D), jnp.float32)),
        )(h_lo, h_hi, agg, agg, Wh, Wa, b.reshape(1, D))
    We, Wo, be, bo = _split_agg_weights(W_agg, b_agg)
    return pl.pallas_call(
        _layer_fused_body,
        grid=(NP // RB,),
        in_specs=[half, half, _lo_spec, _hi_spec, _matf_spec, _matf_spec,
                  _biasf_spec, _mat_spec, _mat_spec, _bias_spec, _bias_spec],
        out_specs=(half, half, pl.BlockSpec((RB, D), lambda i: (i, 0))),
        out_shape=(jax.ShapeDtypeStruct((NP, D), jnp.float32),
                   jax.ShapeDtypeStruct((NP, D), jnp.float32),
                   jax.ShapeDtypeStruct((NPP, D), jnp.int32)),
    )(h_lo, h_hi, agg, agg, Wh, Wa, b.reshape(1, D), We, Wo, be, bo)


def _from_words(aggf):
    a = lax.bitcast_convert_type(aggf.reshape(NW * SEG, DW), jnp.bfloat16)
    return a.reshape(NW * SEG, D)[:N]


def kernel(x, edge_index, W_agg, b_agg, W1, b1, W2, b2):
    src = edge_index[0]
    dst = edge_index[1]

    msrc, mrel, cnt = _make_preprocess()(src, dst)
    segmax = _make_segmax()

    t1 = _dense_pre(x, W_agg, b_agg)
    agg1 = _from_words(segmax(t1, msrc, mrel, cnt))
    h1lo, h1hi, t2 = _dense_layer(x[:NP], x[NP:], agg1, W1, b1, W_agg, b_agg)
    agg2 = _from_words(segmax(t2, msrc, mrel, cnt))
    h2lo, h2hi = _dense_layer(h1lo, h1hi, agg2, W2, b2)
    return jnp.concatenate([h2lo, h2hi], axis=0)


# speedup vs baseline: 1.2198x; 1.2198x over previous
import functools

import jax
import jax.numpy as jnp
from jax import lax
from jax.experimental import pallas as pl
from jax.experimental.pallas import tpu as pltpu
from jax.experimental.pallas import tpu_sc as plsc

N = 10000
E = 320000
D = 128

NW = 32
SEG = 320
LANES = 16
ECHUNK = 3200
NCHUNK = E // ECHUNK
SUBCAP = 1024
CAP = LANES * SUBCAP
GB = 128
LCAP = CAP + GB
PACK = 512
DW = D // 2
NP = N // 2

_sc_params = pltpu.CompilerParams(needs_layout_passes=False)


def _sc_mesh():
    return plsc.VectorSubcoreMesh(core_axis_name="c", subcore_axis_name="s")


def _wid():
    return lax.axis_index("s") * 2 + lax.axis_index("c")


def _make_preprocess():
    @functools.partial(
        pl.kernel,
        mesh=_sc_mesh(),
        compiler_params=_sc_params,
        out_type=(
            jax.ShapeDtypeStruct((NW * LCAP,), jnp.int32),
            jax.ShapeDtypeStruct((NW * LCAP,), jnp.int32),
            jax.ShapeDtypeStruct((NW * LANES,), jnp.int32),
        ),
        scratch_types=[
            pltpu.VMEM((ECHUNK,), jnp.int32),
            pltpu.VMEM((ECHUNK,), jnp.int32),
            pltpu.VMEM((ECHUNK,), jnp.int32),
            pltpu.VMEM((ECHUNK,), jnp.int32),
            pltpu.VMEM((CAP,), jnp.int32),
            pltpu.VMEM((LCAP,), jnp.int32),
            pltpu.VMEM((LCAP,), jnp.int32),
            pltpu.VMEM((LANES,), jnp.int32),
            pltpu.SemaphoreType.DMA,
            pltpu.SemaphoreType.DMA,
        ],
    )
    def preprocess(src_hbm, dst_hbm, msrc_hbm, mrel_hbm, cnt_hbm,
                   src0, dst0, src1, dst1, packed, fsrc, frel, cntv,
                   semA, semB):
        wid = _wid()
        base = wid * SEG
        lane = lax.iota(jnp.int32, LANES)
        ones = jnp.ones((LANES,), jnp.int32)

        padv = jnp.full((LANES,), SEG, jnp.int32)
        zerov = jnp.zeros((LANES,), jnp.int32)

        def fill_packed(i, _):
            packed[pl.ds(i * LANES, LANES)] = padv
            return 0

        lax.fori_loop(0, CAP // LANES, fill_packed, 0, unroll=8)

        def fill_flat(i, _):
            fsrc[pl.ds(i * LANES, LANES)] = zerov
            frel[pl.ds(i * LANES, LANES)] = padv
            return 0

        lax.fori_loop(0, LCAP // LANES, fill_flat, 0, unroll=8)

        def scan_chunk(sbuf, dbuf, counts):
            def vbody(v, counts):
                d = dbuf[pl.ds(v * LANES, LANES)]
                s = sbuf[pl.ds(v * LANES, LANES)]
                rel = d - base
                m = (rel >= 0) & (rel < SEG) & (counts < SUBCAP)
                pos = lane * SUBCAP + counts
                plsc.store_scatter(packed, [pos], s * PACK + rel, mask=m)
                return counts + jnp.where(m, ones, 0)

            return lax.fori_loop(0, ECHUNK // LANES, vbody, counts, unroll=4)

        def start(c, sbuf, dbuf, sem):
            cs = pltpu.async_copy(src_hbm.at[pl.ds(c * ECHUNK, ECHUNK)], sbuf, sem)
            cd = pltpu.async_copy(dst_hbm.at[pl.ds(c * ECHUNK, ECHUNK)], dbuf, sem)
            return cs, cd

        def wait(sbuf, dbuf, sem):
            pltpu.make_async_copy(src_hbm.at[pl.ds(0, ECHUNK)], sbuf, sem).wait()
            pltpu.make_async_copy(dst_hbm.at[pl.ds(0, ECHUNK)], dbuf, sem).wait()

        start(0, src0, dst0, semA)

        def pair_body(i, counts):
            c0 = 2 * i
            wait(src0, dst0, semA)
            start(c0 + 1, src1, dst1, semB)
            counts = scan_chunk(src0, dst0, counts)

            wait(src1, dst1, semB)

            @pl.when(c0 + 2 < NCHUNK)
            def _():
                start(c0 + 2, src0, dst0, semA)

            counts = scan_chunk(src1, dst1, counts)
            return counts

        counts = lax.fori_loop(0, NCHUNK // 2, pair_body,
                               jnp.zeros((LANES,), jnp.int32))

        total = jnp.int32(0)
        for l in range(LANES):
            cl = counts[l]

            def cbody(i, _, l=l, total=total):
                pv = packed[pl.ds(l * SUBCAP + i * LANES, LANES)]
                s = pv // PACK
                hi = (s >= NP).astype(jnp.int32)
                fsrc[pl.ds(total + i * LANES, LANES)] = s - hi * NP
                frel[pl.ds(total + i * LANES, LANES)] = pv % PACK + hi * PACK
                return 0

            lax.fori_loop(0, (cl + LANES - 1) // LANES, cbody, 0)
            total = total + cl

        pltpu.sync_copy(fsrc, msrc_hbm.at[pl.ds(wid * LCAP, LCAP)])
        pltpu.sync_copy(frel, mrel_hbm.at[pl.ds(wid * LCAP, LCAP)])
        cntv[...] = jnp.full((LANES,), total, jnp.int32)
        pltpu.sync_copy(cntv, cnt_hbm.at[pl.ds(wid * LANES, LANES)])

    return preprocess


def _make_segmax():
    @functools.partial(
        pl.kernel,
        mesh=_sc_mesh(),
        compiler_params=_sc_params,
        out_type=jax.ShapeDtypeStruct((NW * SEG * DW,), jnp.int32),
        scratch_types=[
            pltpu.VMEM(((SEG + 1) * DW,), jnp.int32),
            pltpu.VMEM((LCAP,), jnp.int32),
            pltpu.VMEM((LCAP,), jnp.int32),
            pltpu.VMEM((GB, D), jnp.int32),
            pltpu.VMEM((GB, D), jnp.int32),
            pltpu.VMEM((LANES,), jnp.int32),
            pltpu.SemaphoreType.DMA,
            pltpu.SemaphoreType.DMA,
        ],
    )
    def segmax(t_hbm, msrc_hbm, mrel_hbm, cnt_hbm, agg_hbm,
               acc, idxv, relv, rows0, rows1, cntv, semA, semB):
        wid = _wid()

        zrow = jnp.zeros((LANES,), jnp.int32)

        def zbody(i, _):
            acc[pl.ds(i * LANES, LANES)] = zrow
            return 0

        lax.fori_loop(0, (SEG + 1) * DW // LANES, zbody, 0, unroll=8)

        pltpu.sync_copy(cnt_hbm.at[pl.ds(wid * LANES, LANES)], cntv)
        pltpu.sync_copy(msrc_hbm.at[pl.ds(wid * LCAP, LCAP)], idxv)
        pltpu.sync_copy(mrel_hbm.at[pl.ds(wid * LCAP, LCAP)], relv)
        p = cntv[...][0]
        nbatch = (p + (GB - 1)) // GB

        def start(b, rows, sem):
            return pltpu.async_copy(t_hbm.at[idxv.at[pl.ds(b * GB, GB)]],
                                    rows, sem)

        def wait(rows, sem):
            pltpu.make_async_copy(t_hbm.at[idxv.at[pl.ds(0, GB)]], rows,
                                  sem).wait()

        def process(b, rows):
            def gbody(g, _):
                relvec = relv[pl.ds(b * GB + g * LANES, LANES)]
                for l in range(LANES):
                    rp = relvec[l]
                    a0 = (rp % PACK) * DW
                    ho = (rp // PACK) * DW
                    r0 = g * LANES + l
                    for cb in range(DW // LANES):
                        av = plsc.bitcast(acc[pl.ds(a0 + cb * LANES, LANES)],
                                          jnp.bfloat16)
                        rv = plsc.bitcast(rows[r0, pl.ds(ho + cb * LANES, LANES)],
                                          jnp.bfloat16)
                        acc[pl.ds(a0 + cb * LANES, LANES)] = plsc.bitcast(
                            jnp.maximum(av, rv), jnp.int32)
                return 0

            lax.fori_loop(0, GB // LANES, gbody, 0)

        @pl.when(nbatch > 0)
        def _():
            start(0, rows0, semA)

        def pair_body(i, _):
            b0 = 2 * i
            wait(rows0, semA)

            @pl.when(b0 + 1 < nbatch)
            def _():
                start(b0 + 1, rows1, semB)

            process(b0, rows0)

            @pl.when(b0 + 1 < nbatch)
            def _():
                wait(rows1, semB)

                @pl.when(b0 + 2 < nbatch)
                def _():
                    start(b0 + 2, rows0, semA)

                process(b0 + 1, rows1)

            return 0

        lax.fori_loop(0, (nbatch + 1) // 2, pair_body, 0)
        pltpu.sync_copy(acc.at[pl.ds(0, SEG * DW)],
                        agg_hbm.at[pl.ds(wid * SEG * DW, SEG * DW)])

    return segmax


RB = 1000


def _pack16(ze, zo):
    ue = lax.bitcast_convert_type(ze.astype(jnp.bfloat16), jnp.uint16)
    uo = lax.bitcast_convert_type(zo.astype(jnp.bfloat16), jnp.uint16)
    w = ue.astype(jnp.uint32) | (uo.astype(jnp.uint32) << 16)
    return w.astype(jnp.int32)


def _agg_mlp_words(h, we_ref, wo_ref, be_ref, bo_ref):
    ze = jax.nn.relu(
        jnp.dot(h, we_ref[...], preferred_element_type=jnp.float32) + be_ref[...])
    zo = jax.nn.relu(
        jnp.dot(h, wo_ref[...], preferred_element_type=jnp.float32) + bo_ref[...])
    return _pack16(ze, zo)


def _pre_body(hlo_ref, hhi_ref, we_ref, wo_ref, be_ref, bo_ref, out_ref):
    wlo = _agg_mlp_words(hlo_ref[...], we_ref, wo_ref, be_ref, bo_ref)
    whi = _agg_mlp_words(hhi_ref[...], we_ref, wo_ref, be_ref, bo_ref)
    out_ref[...] = jnp.concatenate([wlo, whi], axis=1)


_half_spec = pl.BlockSpec((RB, D), lambda i: (i, 0))
_mat_spec = pl.BlockSpec((D, DW), lambda i: (0, 0))
_matf_spec = pl.BlockSpec((D, D), lambda i: (0, 0))
_bias_spec = pl.BlockSpec((1, DW), lambda i: (0, 0))
_biasf_spec = pl.BlockSpec((1, D), lambda i: (0, 0))
_lo_spec = pl.BlockSpec((RB, D), lambda i: (i, 0))
_hi_spec = pl.BlockSpec((RB, D), lambda i: (i + NP // RB, 0))


def _split_agg_weights(W_agg, b_agg):
    We, Wo = W_agg[:, 0::2], W_agg[:, 1::2]
    be, bo = b_agg[0::2].reshape(1, DW), b_agg[1::2].reshape(1, DW)
    return We, Wo, be, bo


def _dense_pre(h, W_agg, b_agg):
    We, Wo, be, bo = _split_agg_weights(W_agg, b_agg)
    return pl.pallas_call(
        _pre_body,
        grid=(NP // RB,),
        in_specs=[_lo_spec, _hi_spec, _mat_spec, _mat_spec, _bias_spec,
                  _bias_spec],
        out_specs=pl.BlockSpec((RB, D), lambda i: (i, 0)),
        out_shape=jax.ShapeDtypeStruct((NP, D), jnp.int32),
    )(h, h, We, Wo, be, bo)


def _half_update(h, agg, wh_ref, wa_ref, b_ref):
    z = (
        jnp.dot(h, wh_ref[...], preferred_element_type=jnp.float32)
        + jnp.dot(agg, wa_ref[...], preferred_element_type=jnp.float32)
        + b_ref[...]
    )
    z = jax.nn.relu(z)
    norm = jnp.sqrt(jnp.sum(z * z, axis=1, keepdims=True))
    return z / jnp.maximum(norm, 1e-12)


def _layer_fused_body(hlo_ref, hhi_ref, alo_ref, ahi_ref, wh_ref, wa_ref,
                      b_ref, we_ref, wo_ref, be_ref, bo_ref,
                      olo_ref, ohi_ref, t_ref):
    hlo = _half_update(hlo_ref[...], alo_ref[...], wh_ref, wa_ref, b_ref)
    hhi = _half_update(hhi_ref[...], ahi_ref[...], wh_ref, wa_ref, b_ref)
    olo_ref[...] = hlo
    ohi_ref[...] = hhi
    wlo = _agg_mlp_words(hlo, we_ref, wo_ref, be_ref, bo_ref)
    whi = _agg_mlp_words(hhi, we_ref, wo_ref, be_ref, bo_ref)
    t_ref[...] = jnp.concatenate([wlo, whi], axis=1)


def _layer_body(hlo_ref, hhi_ref, alo_ref, ahi_ref, wh_ref, wa_ref, b_ref,
                olo_ref, ohi_ref):
    olo_ref[...] = _half_update(hlo_ref[...], alo_ref[...], wh_ref, wa_ref,
                                b_ref)
    ohi_ref[...] = _half_update(hhi_ref[...], ahi_ref[...], wh_ref, wa_ref,
                                b_ref)


def _dense_layer(h_lo, h_hi, agg, W, b, W_agg=None, b_agg=None):
    Wh, Wa = W[:D], W[D:].astype(jnp.bfloat16)
    half = pl.BlockSpec((RB, D), lambda i: (i, 0))
    if W_agg is None:
        return pl.pallas_call(
            _layer_body,
            grid=(NP // RB,),
            in_specs=[half, half, _lo_spec, _hi_spec, _matf_spec, _matf_spec,
                      _biasf_spec],
            out_specs=(half, half),
            out_shape=(jax.ShapeDtypeStruct((NP, D), jnp.float32),
                       jax.ShapeDtypeStruct((NP, D), jnp.float32)),
        )(h_lo, h_hi, agg, agg, Wh, Wa, b.reshape(1, D))
    We, Wo, be, bo = _split_agg_weights(W_agg, b_agg)
    return pl.pallas_call(
        _layer_fused_body,
        grid=(NP // RB,),
        in_specs=[half, half, _lo_spec, _hi_spec, _matf_spec, _matf_spec,
                  _biasf_spec, _mat_spec, _mat_spec, _bias_spec, _bias_spec],
        out_specs=(half, half, pl.BlockSpec((RB, D), lambda i: (i, 0))),
        out_shape=(jax.ShapeDtypeStruct((NP, D), jnp.float32),
                   jax.ShapeDtypeStruct((NP, D), jnp.float32),
                   jax.ShapeDtypeStruct((NP, D), jnp.int32)),
    )(h_lo, h_hi, agg, agg, Wh, Wa, b.reshape(1, D), We, Wo, be, bo)


def _from_words(aggf):
    a = lax.bitcast_convert_type(aggf.reshape(NW * SEG, DW), jnp.bfloat16)
    return a.reshape(NW * SEG, D)[:N]


def kernel(x, edge_index, W_agg, b_agg, W1, b1, W2, b2):
    src = edge_index[0]
    dst = edge_index[1]

    msrc, mrel, cnt = _make_preprocess()(src, dst)
    segmax = _make_segmax()

    t1 = _dense_pre(x, W_agg, b_agg)
    agg1 = _from_words(segmax(t1, msrc, mrel, cnt))
    h1lo, h1hi, t2 = _dense_layer(x[:NP], x[NP:], agg1, W1, b1, W_agg, b_agg)
    agg2 = _from_words(segmax(t2, msrc, mrel, cnt))
    h2lo, h2hi = _dense_layer(h1lo, h1hi, agg2, W2, b2)
    return jnp.concatenate([h2lo, h2hi], axis=0)
